# baseline (speedup 1.0000x reference)
import functools

import jax
import jax.numpy as jnp
from jax import lax
from jax.experimental import pallas as pl
from jax.experimental.pallas import tpu as pltpu
from jax.experimental.pallas import tpu_sc as plsc

_CHUNK = 128


@functools.lru_cache(maxsize=None)
def _build(B, F):
    info = plsc.get_sparse_core_info()
    NC, NS, L = info.num_cores, info.num_subcores, info.num_lanes
    NW = NC * NS
    assert B % NW == 0 and F % L == 0
    b_per_w = B // NW
    n_chunks = b_per_w // _CHUNK
    n_groups = b_per_w // L

    mesh = plsc.VectorSubcoreMesh(core_axis_name="c", subcore_axis_name="s")

    @functools.partial(
        pl.kernel,
        mesh=mesh,
        out_type=jax.ShapeDtypeStruct((B,), jnp.float32),
        compiler_params=pltpu.CompilerParams(
            needs_layout_passes=False, use_tc_tiling_on_sc=False
        ),
        scratch_types=[
            pltpu.VMEM((b_per_w,), jnp.int32),
            pltpu.VMEM((b_per_w,), jnp.int32),
            pltpu.VMEM((b_per_w, F), jnp.float32),
            pltpu.VMEM((b_per_w, F), jnp.float32),
            pltpu.VMEM((b_per_w,), jnp.float32),
            pltpu.VMEM((b_per_w,), jnp.float32),
            pltpu.VMEM((b_per_w,), jnp.float32),
            pltpu.SemaphoreType.DMA,
        ],
    )
    def k(u_emb_h, i_emb_h, ub_h, ib_h, uidx_h, iidx_h, out_h,
          uidx_v, iidx_v, urows, irows, ubv, ibv, outv, sem):
        wid = lax.axis_index("s") * NC + lax.axis_index("c")
        base = wid * b_per_w
        pltpu.sync_copy(uidx_h.at[pl.ds(base, b_per_w)], uidx_v)
        pltpu.sync_copy(iidx_h.at[pl.ds(base, b_per_w)], iidx_v)
        copies = []
        for c in range(n_chunks):
            s = pl.ds(c * _CHUNK, _CHUNK)
            copies.append(pltpu.async_copy(u_emb_h.at[uidx_v.at[s]], urows.at[s], sem))
            copies.append(pltpu.async_copy(i_emb_h.at[iidx_v.at[s]], irows.at[s], sem))
            copies.append(pltpu.async_copy(ub_h.at[uidx_v.at[s]], ubv.at[s], sem))
            copies.append(pltpu.async_copy(ib_h.at[iidx_v.at[s]], ibv.at[s], sem))
        for cp in copies:
            cp.wait()

        lanes = lax.iota(jnp.int32, L)

        def group(g, carry):
            r0 = g * L
            rows = r0 + lanes
            acc = ubv[pl.ds(r0, L)] + ibv[pl.ds(r0, L)]
            for f in range(F):
                cols = jnp.bitwise_and(f + lanes, F - 1)
                ug = plsc.load_gather(urows, [rows, cols])
                ig = plsc.load_gather(irows, [rows, cols])
                acc = acc + ug * ig
            outv[pl.ds(r0, L)] = acc
            return carry

        lax.fori_loop(0, n_groups, group, 0)
        pltpu.sync_copy(outv, out_h.at[pl.ds(base, b_per_w)])

    return k


def kernel(u_emb, i_emb, u_bias, i_bias, u_idx, i_idx):
    B = u_idx.shape[0]
    F = u_emb.shape[1]
    k = _build(B, F)
    return k(
        u_emb,
        i_emb,
        u_bias.reshape(-1),
        i_bias.reshape(-1),
        u_idx.astype(jnp.int32),
        i_idx.astype(jnp.int32),
    )
